# baseline (speedup 1.0000x reference)
import jax
import jax.numpy as jnp
from jax.experimental import pallas as pl
from jax.experimental.pallas import tpu as pltpu

_B = 4096
_K = 200
_D = 64
_THR = 0.5
_EPS = 1e-12
_BB = 128
_NB = _B // _BB


def _row_block_kernel(q_ref, it_ref, bank_ref, out_bl_ref, out_sh_ref,
                      bsum_ref):
    pi = pl.program_id(0)

    q = q_ref[...]
    it = it_ref[...]
    bank = bank_ref[...]
    bank_t = jnp.transpose(bank, (0, 2, 1))

    it2 = jnp.sum(it * it, axis=-1, keepdims=True)
    q2 = jnp.sum(q * q, axis=-1, keepdims=True)
    qdoti = jnp.sum(q * it, axis=-1, keepdims=True)
    inv_q = 1.0 / jnp.clip(jnp.sqrt(q2), _EPS)
    inv_i = 1.0 / jnp.clip(jnp.sqrt(it2), _EPS)

    ns = jnp.sum(bank_t * bank_t, axis=1)
    dots_q = jnp.sum(bank_t * q[:, :, None], axis=1)
    dots_i = jnp.sum(bank_t * it[:, :, None], axis=1)

    inv_b = 1.0 / jnp.clip(jnp.sqrt(ns), _EPS)
    s_item = dots_i * inv_b * inv_i
    s_query = dots_q * inv_b * inv_q

    best = jnp.max(s_item, axis=-1, keepdims=True)
    kio = jax.lax.broadcasted_iota(jnp.int32, s_item.shape, 1)
    is_max = s_item == best
    j = jnp.min(jnp.where(is_max, kio, _K), axis=-1, keepdims=True)
    oh = kio == j
    ohf = oh.astype(jnp.float32)

    ns_j = jnp.sum(ns * ohf, axis=-1, keepdims=True)
    dq_j = jnp.sum(dots_q * ohf, axis=-1, keepdims=True)
    di_j = jnp.sum(dots_i * ohf, axis=-1, keepdims=True)

    m2 = 0.25 * (ns_j + 2.0 * di_j + it2)
    qdotm = 0.5 * (dq_j + qdoti)
    sim_j = qdotm * inv_q / jnp.clip(jnp.sqrt(m2), _EPS)
    qi_cos = qdoti * inv_q * inv_i

    m_ = jnp.max(s_query, axis=-1, keepdims=True)
    e_q = jnp.exp(s_query - m_)
    s_sum = jnp.sum(e_q, axis=-1, keepdims=True)
    e_q0 = e_q[:, 0:1]
    e_qj = jnp.sum(e_q * ohf, axis=-1, keepdims=True)
    e_blj = jnp.exp(sim_j - m_)
    e_shit = jnp.exp(qi_cos - m_)
    z_bl = s_sum - e_qj + e_blj
    z_sh = s_sum - e_q0 + e_shit

    u = jnp.sum(e_q[:, :, None] * bank, axis=1)
    bank_j = jnp.sum(ohf[:, :, None] * bank, axis=1)
    bank0 = bank[:, 0, :]

    out_bl_ref[...] = (u + (0.5 * e_blj - e_qj) * bank_j
                       + (0.5 * e_blj) * it) / z_bl
    out_sh_ref[...] = (u - e_q0 * bank0 + e_shit * it) / z_sh

    @pl.when(pi == 0)
    def _init():
        bsum_ref[...] = jnp.zeros_like(bsum_ref)

    bsum_ref[...] += jnp.sum(best).reshape(1, 1)


@jax.jit
def kernel(query, item, memory_bank):
    grid = (_NB,)
    out_bl, out_sh, bsum = pl.pallas_call(
        _row_block_kernel,
        grid=grid,
        in_specs=[
            pl.BlockSpec((_BB, _D), lambda i: (i, 0)),
            pl.BlockSpec((_BB, _D), lambda i: (i, 0)),
            pl.BlockSpec((_BB, _K, _D), lambda i: (i, 0, 0)),
        ],
        out_specs=[
            pl.BlockSpec((_BB, _D), lambda i: (i, 0)),
            pl.BlockSpec((_BB, _D), lambda i: (i, 0)),
            pl.BlockSpec((1, 1), lambda i: (0, 0)),
        ],
        out_shape=[
            jax.ShapeDtypeStruct((_B, _D), jnp.float32),
            jax.ShapeDtypeStruct((_B, _D), jnp.float32),
            jax.ShapeDtypeStruct((1, 1), jnp.float32),
        ],
    )(query, item, memory_bank)
    mean_best = bsum[0, 0] / _B
    return jnp.where(mean_best >= _THR, out_bl, out_sh)


# speedup vs baseline: 1.1643x; 1.1643x over previous
import jax
import jax.numpy as jnp
from jax.experimental import pallas as pl
from jax.experimental.pallas import tpu as pltpu

_B = 4096
_K = 200
_D = 64
_THR = 0.5
_EPS = 1e-12
_BB = 128
_NB = _B // _BB


def _two_phase_kernel(q_ref, it_ref, bank_ref, out_ref,
                      eq_ref, j_ref, coef_ref, bsum_ref):
    s = pl.program_id(0)

    @pl.when(s < _NB)
    def _phase_a():
        q = q_ref[...]
        it = it_ref[...]
        bank = bank_ref[...]

        it2 = jnp.sum(it * it, axis=-1, keepdims=True)
        q2 = jnp.sum(q * q, axis=-1, keepdims=True)
        qdoti = jnp.sum(q * it, axis=-1, keepdims=True)
        inv_q = 1.0 / jnp.clip(jnp.sqrt(q2), _EPS)
        inv_i = 1.0 / jnp.clip(jnp.sqrt(it2), _EPS)

        bank_t = jnp.transpose(bank, (0, 2, 1))
        ns = jnp.sum(bank_t * bank_t, axis=1)
        dots_q = jnp.sum(bank_t * q[:, :, None], axis=1)
        dots_i = jnp.sum(bank_t * it[:, :, None], axis=1)

        inv_b = 1.0 / jnp.clip(jnp.sqrt(ns), _EPS)
        s_item = dots_i * inv_b * inv_i
        s_query = dots_q * inv_b * inv_q

        best = jnp.max(s_item, axis=-1, keepdims=True)
        kio = jax.lax.broadcasted_iota(jnp.int32, s_item.shape, 1)
        is_max = s_item == best
        j = jnp.min(jnp.where(is_max, kio, _K), axis=-1, keepdims=True)
        oh = kio == j
        ohf = oh.astype(jnp.float32)

        ns_j = jnp.sum(ns * ohf, axis=-1, keepdims=True)
        dq_j = jnp.sum(dots_q * ohf, axis=-1, keepdims=True)
        di_j = jnp.sum(dots_i * ohf, axis=-1, keepdims=True)

        m2 = 0.25 * (ns_j + 2.0 * di_j + it2)
        qdotm = 0.5 * (dq_j + qdoti)
        sim_j = qdotm * inv_q / jnp.clip(jnp.sqrt(m2), _EPS)
        qi_cos = qdoti * inv_q * inv_i

        m_ = jnp.max(s_query, axis=-1, keepdims=True)
        e_q = jnp.exp(s_query - m_)
        s_sum = jnp.sum(e_q, axis=-1, keepdims=True)
        e_q0 = e_q[:, 0:1]
        e_qj = jnp.sum(e_q * ohf, axis=-1, keepdims=True)
        e_blj = jnp.exp(sim_j - m_)
        e_shit = jnp.exp(qi_cos - m_)
        inv_zbl = 1.0 / (s_sum - e_qj + e_blj)
        inv_zsh = 1.0 / (s_sum - e_q0 + e_shit)

        row0 = s * _BB
        eq_ref[pl.ds(row0, _BB), :] = e_q
        j_ref[pl.ds(row0, _BB), :] = j
        coef_ref[pl.ds(row0, _BB), 0:1] = inv_zbl
        coef_ref[pl.ds(row0, _BB), 1:2] = inv_zsh
        coef_ref[pl.ds(row0, _BB), 2:3] = (0.5 * e_blj - e_qj) * inv_zbl
        coef_ref[pl.ds(row0, _BB), 3:4] = (0.5 * e_blj) * inv_zbl
        coef_ref[pl.ds(row0, _BB), 4:5] = e_shit * inv_zsh

        @pl.when(s == 0)
        def _init():
            bsum_ref[...] = jnp.zeros_like(bsum_ref)

        bsum_ref[...] += jnp.sum(best).reshape(1, 1)

    @pl.when(s >= _NB)
    def _phase_b():
        row0 = (s - _NB) * _BB
        sel = bsum_ref[0, 0] >= _THR * _B
        e_q = eq_ref[pl.ds(row0, _BB), :]
        j = j_ref[pl.ds(row0, _BB), :]
        kio = jax.lax.broadcasted_iota(jnp.int32, e_q.shape, 1)
        ohf = (kio == j).astype(jnp.float32)
        nz0 = (kio != 0).astype(jnp.float32)
        inv_zbl = coef_ref[pl.ds(row0, _BB), 0:1]
        inv_zsh = coef_ref[pl.ds(row0, _BB), 1:2]
        dl = coef_ref[pl.ds(row0, _BB), 2:3]
        cbl = coef_ref[pl.ds(row0, _BB), 3:4]
        csh = coef_ref[pl.ds(row0, _BB), 4:5]
        w = jnp.where(sel, e_q * inv_zbl + dl * ohf, e_q * inv_zsh * nz0)
        c = jnp.where(sel, cbl, csh)
        bank = bank_ref[...]
        it = it_ref[...]
        out_ref[...] = jnp.sum(w[:, :, None] * bank, axis=1) + c * it


@jax.jit
def kernel(query, item, memory_bank):
    grid = (2 * _NB,)
    blk = lambda s: (jnp.where(s < _NB, s, s - _NB), 0)
    blk3 = lambda s: (jnp.where(s < _NB, s, s - _NB), 0, 0)
    out = pl.pallas_call(
        _two_phase_kernel,
        grid=grid,
        in_specs=[
            pl.BlockSpec((_BB, _D), blk),
            pl.BlockSpec((_BB, _D), blk),
            pl.BlockSpec((_BB, _K, _D), blk3),
        ],
        out_specs=pl.BlockSpec((_BB, _D), lambda s: (jnp.maximum(s - _NB, 0), 0)),
        out_shape=jax.ShapeDtypeStruct((_B, _D), jnp.float32),
        scratch_shapes=[
            pltpu.VMEM((_B, _K), jnp.float32),
            pltpu.VMEM((_B, 1), jnp.int32),
            pltpu.VMEM((_B, 8), jnp.float32),
            pltpu.VMEM((1, 1), jnp.float32),
        ],
    )(query, item, memory_bank)
    return out
